# baseline (speedup 1.0000x reference)
import dataclasses
import functools

import numpy as np
import jax
import jax.numpy as jnp
from jax import lax
from jax.experimental import pallas as pl
from jax.experimental.pallas import tpu as pltpu
from jax.experimental.pallas import tpu_sc as plsc

_N = 10000
_E = 320000
_D = 128
_C = 64

_NC = 2
_NS = 16
_NW = _NC * _NS
_B = 128
_NB = 80
_EPAD = _NW * _NB * _B
_R = 10112
_RZ = _R // _NS

def _mesh():
    return plsc.VectorSubcoreMesh(core_axis_name="c", subcore_axis_name="s")


_CNB = 16
_NCH = _NB // _CNB

_P = np.empty(_D, np.int32)
for _g in range(_D // 32):
    for _i in range(16):
        _P[32 * _g + 2 * _i] = 32 * _g + _i
        _P[32 * _g + 2 * _i + 1] = 32 * _g + 16 + _i


def _make_agg():
    out_type = jax.ShapeDtypeStruct((_NC, _R, _D), jnp.float32)
    scratch = [
        pltpu.VMEM((_CNB, _B), jnp.int32),
        pltpu.VMEM((_CNB, _B), jnp.int32),
        pltpu.VMEM((_B, _D // 2), jnp.int32),
        pltpu.VMEM((_B, _D // 2), jnp.int32),
        pltpu.VMEM((_B, _D), jnp.float32),
        pltpu.VMEM_SHARED((_R, _D), jnp.float32),
        pltpu.SemaphoreType.DMA,
        pltpu.SemaphoreType.DMA,
        pltpu.SemaphoreType.DMA,
    ]

    def body(h_hbm, src_hbm, dst_hbm, z_hbm, out_hbm,
             src_c, dst_c, b0, b1, fstage, acc_sh, gs0, gs1, ss):
        c = lax.axis_index("c")
        s = lax.axis_index("s")
        w = s * _NC + c

        pltpu.sync_copy(z_hbm.at[pl.ds(s * _RZ, _RZ)],
                        acc_sh.at[pl.ds(s * _RZ, _RZ)])
        plsc.subcore_barrier()

        def gather(idx, buf, sem):
            pltpu.async_copy(h_hbm.at[idx], buf, sem)

        def gwait(buf, sem):
            pltpu.make_async_copy(h_hbm.at[src_c.at[0]], buf, sem).wait()

        def scat(idx):
            pltpu.async_copy(fstage, acc_sh.at[idx], ss, add=True)

        def swait():
            pltpu.make_async_copy(fstage, acc_sh.at[dst_c.at[0]], ss).wait()

        def convert(bbuf):
            @plsc.parallel_loop(0, _B, unroll=2)
            def _(r):
                for k in range(_D // 32):
                    pair = plsc.bitcast(bbuf[r, pl.ds(16 * k, 16)],
                                        jnp.bfloat16)
                    lo, hi = plsc.unpack(
                        pair, format=plsc.PackFormat.INTERLEAVED,
                        preferred_element_type=jnp.float32)
                    fstage[r, pl.ds(32 * k, 16)] = lo
                    fstage[r, pl.ds(32 * k + 16, 16)] = hi

        @pl.loop(0, _NCH)
        def _(ch):
            base = w * _NB + ch * _CNB
            pltpu.sync_copy(src_hbm.at[pl.ds(base, _CNB)], src_c)
            pltpu.sync_copy(dst_hbm.at[pl.ds(base, _CNB)], dst_c)
            gather(src_c.at[0], b0, gs0)
            gather(src_c.at[1], b1, gs1)

            @pl.loop(0, _CNB, step=2)
            def _(j):
                gwait(b0, gs0)

                @pl.when(jnp.logical_not((ch == 0) & (j == 0)))
                def _():
                    swait()

                convert(b0)

                @pl.when(j + 2 < _CNB)
                def _():
                    gather(src_c.at[j + 2], b0, gs0)

                scat(dst_c.at[j])

                gwait(b1, gs1)
                swait()
                convert(b1)

                @pl.when(j + 3 < _CNB)
                def _():
                    gather(src_c.at[j + 3], b1, gs1)

                scat(dst_c.at[j + 1])

        swait()
        plsc.subcore_barrier()
        pltpu.sync_copy(acc_sh.at[pl.ds(s * _RZ, _RZ)],
                        out_hbm.at[c, pl.ds(s * _RZ, _RZ)])

    cp = pltpu.CompilerParams()
    fields = pltpu.CompilerParams.__dataclass_fields__
    if "needs_layout_passes" in fields:
        cp = dataclasses.replace(cp, needs_layout_passes=False)
    if "use_tc_tiling_on_sc" in fields:
        cp = dataclasses.replace(cp, use_tc_tiling_on_sc=False)
    return pl.kernel(body, out_type=out_type, mesh=_mesh(),
                     scratch_types=scratch, compiler_params=cp)


def _make_cnt():
    out_type = jax.ShapeDtypeStruct((_NC, _R, _D), jnp.float32)
    scratch = [
        pltpu.VMEM((_CNB, _B), jnp.int32),
        pltpu.VMEM((_B, _D), jnp.float32),
        pltpu.VMEM_SHARED((_R, _D), jnp.float32),
        pltpu.SemaphoreType.DMA,
        pltpu.SemaphoreType.DMA,
    ]

    def body(dst_hbm, z_hbm, ones_hbm, out_hbm,
             dst_c, ones_v, acc_sh, ss0, ss1):
        c = lax.axis_index("c")
        s = lax.axis_index("s")
        w = s * _NC + c

        pltpu.sync_copy(z_hbm.at[pl.ds(s * _RZ, _RZ)],
                        acc_sh.at[pl.ds(s * _RZ, _RZ)])
        pltpu.sync_copy(ones_hbm, ones_v)
        plsc.subcore_barrier()

        def scat(idx, sem):
            pltpu.async_copy(ones_v, acc_sh.at[idx], sem, add=True)

        def swait(idx, sem):
            pltpu.make_async_copy(ones_v, acc_sh.at[idx], sem).wait()

        @pl.loop(0, _NCH)
        def _(ch):
            base = w * _NB + ch * _CNB
            pltpu.sync_copy(dst_hbm.at[pl.ds(base, _CNB)], dst_c)
            scat(dst_c.at[0], ss0)
            scat(dst_c.at[1], ss1)

            @pl.loop(2, _CNB, step=2)
            def _(j):
                swait(dst_c.at[j], ss0)
                scat(dst_c.at[j], ss0)
                swait(dst_c.at[j + 1], ss1)
                scat(dst_c.at[j + 1], ss1)

            swait(dst_c.at[_CNB - 2], ss0)
            swait(dst_c.at[_CNB - 1], ss1)

        plsc.subcore_barrier()
        pltpu.sync_copy(acc_sh.at[pl.ds(s * _RZ, _RZ)],
                        out_hbm.at[c, pl.ds(s * _RZ, _RZ)])

    return pl.kernel(body, out_type=out_type, mesh=_mesh(),
                     scratch_types=scratch)


_built = {}


def _agg(*args):
    if "agg" not in _built:
        _built["agg"] = _make_agg()
    return _built["agg"](*args)


def _cnt(*args):
    if "cnt" not in _built:
        _built["cnt"] = _make_cnt()
    return _built["cnt"](*args)

_BLK = 1000


def _relu_proj(x, wt, b):
    def body(x_ref, w_ref, b_ref, o_ref):
        o_ref[...] = jnp.maximum(
            jnp.dot(x_ref[...], w_ref[...],
                    preferred_element_type=jnp.float32) + b_ref[...], 0.0)

    return pl.pallas_call(
        body,
        grid=(_N // _BLK,),
        in_specs=[
            pl.BlockSpec((_BLK, _D), lambda i: (i, 0)),
            pl.BlockSpec((_D, _D), lambda i: (0, 0)),
            pl.BlockSpec((1, _D), lambda i: (0, 0)),
        ],
        out_specs=pl.BlockSpec((_BLK, _D), lambda i: (i, 0)),
        out_shape=jax.ShapeDtypeStruct((_N, _D), jnp.float32),
    )(x, wt, b)


def _step(h, aggp, degp, wlt, blv, wrt, g, bt):
    def body(h_ref, a_ref, d_ref, wl_ref, bl_ref, wr_ref, g_ref, bt_ref,
             o_ref):
        hv = h_ref[...]
        aggs = a_ref[0] + a_ref[1]
        degl = d_ref[0] + d_ref[1]
        deg = jnp.sum(degl, axis=1, keepdims=True) * (1.0 / _D)
        agg = aggs / jnp.maximum(deg, 1.0)
        h2 = (jnp.dot(agg, wl_ref[...], preferred_element_type=jnp.float32)
              + jnp.dot(hv, wr_ref[...], preferred_element_type=jnp.float32)
              + bl_ref[...])
        mu = jnp.mean(h2, axis=1, keepdims=True)
        dev = h2 - mu
        var = jnp.mean(dev * dev, axis=1, keepdims=True)
        hn = g_ref[...] * dev * lax.rsqrt(var + 1e-5) + bt_ref[...]
        o_ref[...] = jnp.maximum(hn, 0.0) + hv

    return pl.pallas_call(
        body,
        grid=(_N // _BLK,),
        in_specs=[
            pl.BlockSpec((_BLK, _D), lambda i: (i, 0)),
            pl.BlockSpec((_NC, _BLK, _D), lambda i: (0, i, 0)),
            pl.BlockSpec((_NC, _BLK, _D), lambda i: (0, i, 0)),
            pl.BlockSpec((_D, _D), lambda i: (0, 0)),
            pl.BlockSpec((1, _D), lambda i: (0, 0)),
            pl.BlockSpec((_D, _D), lambda i: (0, 0)),
            pl.BlockSpec((1, _D), lambda i: (0, 0)),
            pl.BlockSpec((1, _D), lambda i: (0, 0)),
        ],
        out_specs=pl.BlockSpec((_BLK, _D), lambda i: (i, 0)),
        out_shape=jax.ShapeDtypeStruct((_N, _D), jnp.float32),
    )(h, aggp, degp, wlt, blv, wrt, g, bt)


def _outproj3(h0, h1, h2, w0, w1, w2, b):
    def body(h0_ref, h1_ref, h2_ref, w0_ref, w1_ref, w2_ref, b_ref, o_ref):
        o_ref[...] = (
            b_ref[...]
            + jnp.dot(h0_ref[...], w0_ref[...],
                      preferred_element_type=jnp.float32)
            + jnp.dot(h1_ref[...], w1_ref[...],
                      preferred_element_type=jnp.float32)
            + jnp.dot(h2_ref[...], w2_ref[...],
                      preferred_element_type=jnp.float32))

    hspec = pl.BlockSpec((_BLK, _D), lambda i: (i, 0))
    wspec = pl.BlockSpec((_D, _C), lambda i: (0, 0))
    return pl.pallas_call(
        body,
        grid=(_N // _BLK,),
        in_specs=[hspec, hspec, hspec, wspec, wspec, wspec,
                  pl.BlockSpec((1, _C), lambda i: (0, 0))],
        out_specs=pl.BlockSpec((_BLK, _C), lambda i: (i, 0)),
        out_shape=jax.ShapeDtypeStruct((_N, _C), jnp.float32),
    )(h0, h1, h2, w0, w1, w2, b)


def _outproj_final(part, h3, w3):
    def body(p_ref, h_ref, w_ref, o_ref):
        o_ref[...] = p_ref[...] + jnp.dot(
            h_ref[...], w_ref[...], preferred_element_type=jnp.float32)

    return pl.pallas_call(
        body,
        grid=(_N // _BLK,),
        in_specs=[
            pl.BlockSpec((_BLK, _C), lambda i: (i, 0)),
            pl.BlockSpec((_BLK, _D), lambda i: (i, 0)),
            pl.BlockSpec((_D, _C), lambda i: (0, 0)),
        ],
        out_specs=pl.BlockSpec((_BLK, _C), lambda i: (i, 0)),
        out_shape=jax.ShapeDtypeStruct((_N, _C), jnp.float32),
    )(part, h3, w3)


_PAD_I = np.arange(_EPAD - _E, dtype=np.int32)
_PAD_SRC = _PAD_I % _N
_PAD_DST = (_N + _PAD_I % (_R - _N)).astype(np.int32)


def kernel(x, edge_index, W_in, b_in, Wl, bl, Wr, gamma, beta, W_out, b_out):
    src = edge_index[0]
    dst = edge_index[1]
    src2d = jnp.concatenate([src, _PAD_SRC]).reshape(_NW * _NB, _B)
    dst2d = jnp.concatenate([dst, _PAD_DST]).reshape(_NW * _NB, _B)
    z = jnp.zeros((_R, _D), jnp.float32)

    def pack_tbl(h):
        hb = h[:, _P].astype(jnp.bfloat16).reshape(_N, _D // 2, 2)
        return lax.bitcast_convert_type(hb, jnp.int32)

    h0 = _relu_proj(x, W_in.T, b_in.reshape(1, _D))
    degp = _cnt(dst2d, z, jnp.ones((_B, _D), jnp.float32))
    agg0 = _agg(pack_tbl(h0), src2d, dst2d, z)
    h1 = _step(h0, agg0, degp, Wl[0].T, bl[0].reshape(1, _D), Wr[0].T,
               gamma[0].reshape(1, _D), beta[0].reshape(1, _D))
    agg1 = _agg(pack_tbl(h1), src2d, dst2d, z)
    h2 = _step(h1, agg1, degp, Wl[1].T, bl[1].reshape(1, _D), Wr[1].T,
               gamma[1].reshape(1, _D), beta[1].reshape(1, _D))
    agg2 = _agg(pack_tbl(h2), src2d, dst2d, z)
    wo = W_out.T
    part = _outproj3(h0, h1, h2, wo[:_D], wo[_D:2 * _D], wo[2 * _D:3 * _D],
                     b_out.reshape(1, _C))
    h3 = _step(h2, agg2, degp, Wl[2].T, bl[2].reshape(1, _D), Wr[2].T,
               gamma[2].reshape(1, _D), beta[2].reshape(1, _D))
    out = _outproj_final(part, h3, wo[3 * _D:])
    return out


# speedup vs baseline: 1.1460x; 1.1460x over previous
import dataclasses
import functools

import numpy as np
import jax
import jax.numpy as jnp
from jax import lax
from jax.experimental import pallas as pl
from jax.experimental.pallas import tpu as pltpu
from jax.experimental.pallas import tpu_sc as plsc

_N = 10000
_E = 320000
_D = 128
_C = 64

_NC = 2
_NS = 16
_NW = _NC * _NS
_B = 128
_NB = 80
_EPAD = _NW * _NB * _B
_R = 10112
_RZ = _R // _NS

def _mesh():
    return plsc.VectorSubcoreMesh(core_axis_name="c", subcore_axis_name="s")


_CNB = 16
_NCH = _NB // _CNB


def _make_agg():
    out_type = jax.ShapeDtypeStruct((_NC, _R, _D), jnp.float32)
    scratch = [
        pltpu.VMEM((_CNB, _B), jnp.int32),
        pltpu.VMEM((_CNB, _B), jnp.int32),
        pltpu.VMEM((_B, _D // 2), jnp.int32),
        pltpu.VMEM((_B, _D // 2), jnp.int32),
        pltpu.VMEM((_B, _D), jnp.float32),
        pltpu.VMEM_SHARED((_R, _D), jnp.float32),
        pltpu.SemaphoreType.DMA,
        pltpu.SemaphoreType.DMA,
        pltpu.SemaphoreType.DMA,
    ]

    def body(h_hbm, src_hbm, dst_hbm, z_hbm, out_hbm,
             src_c, dst_c, b0, b1, fstage, acc_sh, gs0, gs1, ss):
        c = lax.axis_index("c")
        s = lax.axis_index("s")
        w = s * _NC + c

        pltpu.sync_copy(z_hbm.at[pl.ds(s * _RZ, _RZ)],
                        acc_sh.at[pl.ds(s * _RZ, _RZ)])
        plsc.subcore_barrier()

        def gather(idx, buf, sem):
            pltpu.async_copy(h_hbm.at[idx], buf, sem)

        def gwait(buf, sem):
            pltpu.make_async_copy(h_hbm.at[src_c.at[0]], buf, sem).wait()

        def scat(idx):
            pltpu.async_copy(fstage, acc_sh.at[idx], ss, add=True)

        def swait():
            pltpu.make_async_copy(fstage, acc_sh.at[dst_c.at[0]], ss).wait()

        def convert(bbuf):
            @plsc.parallel_loop(0, _B, unroll=2)
            def _(r):
                for k in range(_D // 32):
                    pair = plsc.bitcast(bbuf[r, pl.ds(16 * k, 16)],
                                        jnp.bfloat16)
                    lo, hi = plsc.unpack(
                        pair, format=plsc.PackFormat.INTERLEAVED,
                        preferred_element_type=jnp.float32)
                    fstage[r, pl.ds(16 * k, 16)] = lo
                    fstage[r, pl.ds(_D // 2 + 16 * k, 16)] = hi

        @pl.loop(0, _NCH)
        def _(ch):
            base = w * _NB + ch * _CNB
            pltpu.sync_copy(src_hbm.at[pl.ds(base, _CNB)], src_c)
            pltpu.sync_copy(dst_hbm.at[pl.ds(base, _CNB)], dst_c)
            gather(src_c.at[0], b0, gs0)
            gather(src_c.at[1], b1, gs1)

            @pl.loop(0, _CNB, step=2)
            def _(j):
                gwait(b0, gs0)

                @pl.when(jnp.logical_not((ch == 0) & (j == 0)))
                def _():
                    swait()

                convert(b0)

                @pl.when(j + 2 < _CNB)
                def _():
                    gather(src_c.at[j + 2], b0, gs0)

                scat(dst_c.at[j])

                gwait(b1, gs1)
                swait()
                convert(b1)

                @pl.when(j + 3 < _CNB)
                def _():
                    gather(src_c.at[j + 3], b1, gs1)

                scat(dst_c.at[j + 1])

        swait()
        plsc.subcore_barrier()
        pltpu.sync_copy(acc_sh.at[pl.ds(s * _RZ, _RZ)],
                        out_hbm.at[c, pl.ds(s * _RZ, _RZ)])

    return pl.kernel(body, out_type=out_type, mesh=_mesh(),
                     scratch_types=scratch, compiler_params=_sc_params())


def _sc_params():
    cp = pltpu.CompilerParams()
    fields = pltpu.CompilerParams.__dataclass_fields__
    if "needs_layout_passes" in fields:
        cp = dataclasses.replace(cp, needs_layout_passes=False)
    if "use_tc_tiling_on_sc" in fields:
        cp = dataclasses.replace(cp, use_tc_tiling_on_sc=False)
    return cp


def _make_cnt():
    out_type = jax.ShapeDtypeStruct((_NC, _R, _D), jnp.float32)
    scratch = [
        pltpu.VMEM((_CNB, _B), jnp.int32),
        pltpu.VMEM((_B, _D), jnp.float32),
        pltpu.VMEM_SHARED((_R, _D), jnp.float32),
        pltpu.SemaphoreType.DMA,
        pltpu.SemaphoreType.DMA,
    ]

    def body(dst_hbm, z_hbm, ones_hbm, out_hbm,
             dst_c, ones_v, acc_sh, ss0, ss1):
        c = lax.axis_index("c")
        s = lax.axis_index("s")
        w = s * _NC + c

        pltpu.sync_copy(z_hbm.at[pl.ds(s * _RZ, _RZ)],
                        acc_sh.at[pl.ds(s * _RZ, _RZ)])
        pltpu.sync_copy(ones_hbm, ones_v)
        plsc.subcore_barrier()

        def scat(idx, sem):
            pltpu.async_copy(ones_v, acc_sh.at[idx], sem, add=True)

        def swait(idx, sem):
            pltpu.make_async_copy(ones_v, acc_sh.at[idx], sem).wait()

        @pl.loop(0, _NCH)
        def _(ch):
            base = w * _NB + ch * _CNB
            pltpu.sync_copy(dst_hbm.at[pl.ds(base, _CNB)], dst_c)
            scat(dst_c.at[0], ss0)
            scat(dst_c.at[1], ss1)

            @pl.loop(2, _CNB, step=2)
            def _(j):
                swait(dst_c.at[j], ss0)
                scat(dst_c.at[j], ss0)
                swait(dst_c.at[j + 1], ss1)
                scat(dst_c.at[j + 1], ss1)

            swait(dst_c.at[_CNB - 2], ss0)
            swait(dst_c.at[_CNB - 1], ss1)

        plsc.subcore_barrier()
        pltpu.sync_copy(acc_sh.at[pl.ds(s * _RZ, _RZ)],
                        out_hbm.at[c, pl.ds(s * _RZ, _RZ)])

    return pl.kernel(body, out_type=out_type, mesh=_mesh(),
                     scratch_types=scratch, compiler_params=_sc_params())


_built = {}


def _agg(*args):
    if "agg" not in _built:
        _built["agg"] = _make_agg()
    return _built["agg"](*args)


def _cnt(*args):
    if "cnt" not in _built:
        _built["cnt"] = _make_cnt()
    return _built["cnt"](*args)

_BLK = 1000


def _pack_rows(hv):
    ua = lax.bitcast_convert_type(hv[:, :_D // 2], jnp.uint32)
    ub = lax.bitcast_convert_type(hv[:, _D // 2:], jnp.uint32)
    ra = ua + jnp.uint32(0x7FFF) + ((ua >> 16) & jnp.uint32(1))
    rb = ub + jnp.uint32(0x7FFF) + ((ub >> 16) & jnp.uint32(1))
    w = (ra >> 16) | (rb & jnp.uint32(0xFFFF0000))
    return lax.bitcast_convert_type(w, jnp.int32)


def _relu_proj(x, wt, b):
    def body(x_ref, w_ref, b_ref, o_ref, t_ref):
        h = jnp.maximum(
            jnp.dot(x_ref[...], w_ref[...],
                    preferred_element_type=jnp.float32) + b_ref[...], 0.0)
        o_ref[...] = h
        t_ref[...] = _pack_rows(h)

    return pl.pallas_call(
        body,
        grid=(_N // _BLK,),
        in_specs=[
            pl.BlockSpec((_BLK, _D), lambda i: (i, 0)),
            pl.BlockSpec((_D, _D), lambda i: (0, 0)),
            pl.BlockSpec((1, _D), lambda i: (0, 0)),
        ],
        out_specs=[pl.BlockSpec((_BLK, _D), lambda i: (i, 0)),
                   pl.BlockSpec((_BLK, _D // 2), lambda i: (i, 0))],
        out_shape=[jax.ShapeDtypeStruct((_N, _D), jnp.float32),
                   jax.ShapeDtypeStruct((_N, _D // 2), jnp.int32)],
    )(x, wt, b)


def _step(h, aggp, degp, wlt, blv, wrt, g, bt, pack=True):
    def body(h_ref, a_ref, d_ref, wl_ref, bl_ref, wr_ref, g_ref, bt_ref,
             o_ref, *t_ref):
        hv = h_ref[...]
        aggs = a_ref[0] + a_ref[1]
        degl = d_ref[0] + d_ref[1]
        deg = jnp.sum(degl, axis=1, keepdims=True) * (1.0 / _D)
        agg = aggs / jnp.maximum(deg, 1.0)
        h2 = (jnp.dot(agg, wl_ref[...], preferred_element_type=jnp.float32)
              + jnp.dot(hv, wr_ref[...], preferred_element_type=jnp.float32)
              + bl_ref[...])
        mu = jnp.mean(h2, axis=1, keepdims=True)
        dev = h2 - mu
        var = jnp.mean(dev * dev, axis=1, keepdims=True)
        hn = g_ref[...] * dev * lax.rsqrt(var + 1e-5) + bt_ref[...]
        hnew = jnp.maximum(hn, 0.0) + hv
        o_ref[...] = hnew
        if pack:
            t_ref[0][...] = _pack_rows(hnew)

    out_specs = [pl.BlockSpec((_BLK, _D), lambda i: (i, 0))]
    out_shape = [jax.ShapeDtypeStruct((_N, _D), jnp.float32)]
    if pack:
        out_specs.append(pl.BlockSpec((_BLK, _D // 2), lambda i: (i, 0)))
        out_shape.append(jax.ShapeDtypeStruct((_N, _D // 2), jnp.int32))
    return pl.pallas_call(
        body,
        grid=(_N // _BLK,),
        in_specs=[
            pl.BlockSpec((_BLK, _D), lambda i: (i, 0)),
            pl.BlockSpec((_NC, _BLK, _D), lambda i: (0, i, 0)),
            pl.BlockSpec((_NC, _BLK, _D), lambda i: (0, i, 0)),
            pl.BlockSpec((_D, _D), lambda i: (0, 0)),
            pl.BlockSpec((1, _D), lambda i: (0, 0)),
            pl.BlockSpec((_D, _D), lambda i: (0, 0)),
            pl.BlockSpec((1, _D), lambda i: (0, 0)),
            pl.BlockSpec((1, _D), lambda i: (0, 0)),
        ],
        out_specs=out_specs,
        out_shape=out_shape,
    )(h, aggp, degp, wlt, blv, wrt, g, bt)


def _outproj3(h0, h1, h2, w0, w1, w2, b):
    def body(h0_ref, h1_ref, h2_ref, w0_ref, w1_ref, w2_ref, b_ref, o_ref):
        o_ref[...] = (
            b_ref[...]
            + jnp.dot(h0_ref[...], w0_ref[...],
                      preferred_element_type=jnp.float32)
            + jnp.dot(h1_ref[...], w1_ref[...],
                      preferred_element_type=jnp.float32)
            + jnp.dot(h2_ref[...], w2_ref[...],
                      preferred_element_type=jnp.float32))

    hspec = pl.BlockSpec((_BLK, _D), lambda i: (i, 0))
    wspec = pl.BlockSpec((_D, _C), lambda i: (0, 0))
    return pl.pallas_call(
        body,
        grid=(_N // _BLK,),
        in_specs=[hspec, hspec, hspec, wspec, wspec, wspec,
                  pl.BlockSpec((1, _C), lambda i: (0, 0))],
        out_specs=pl.BlockSpec((_BLK, _C), lambda i: (i, 0)),
        out_shape=jax.ShapeDtypeStruct((_N, _C), jnp.float32),
    )(h0, h1, h2, w0, w1, w2, b)


def _outproj_final(part, h3, w3):
    def body(p_ref, h_ref, w_ref, o_ref):
        o_ref[...] = p_ref[...] + jnp.dot(
            h_ref[...], w_ref[...], preferred_element_type=jnp.float32)

    return pl.pallas_call(
        body,
        grid=(_N // _BLK,),
        in_specs=[
            pl.BlockSpec((_BLK, _C), lambda i: (i, 0)),
            pl.BlockSpec((_BLK, _D), lambda i: (i, 0)),
            pl.BlockSpec((_D, _C), lambda i: (0, 0)),
        ],
        out_specs=pl.BlockSpec((_BLK, _C), lambda i: (i, 0)),
        out_shape=jax.ShapeDtypeStruct((_N, _C), jnp.float32),
    )(part, h3, w3)


_PAD_I = np.arange(_EPAD - _E, dtype=np.int32)
_PAD_SRC = _PAD_I % _N
_PAD_DST = (_N + _PAD_I % (_R - _N)).astype(np.int32)


def kernel(x, edge_index, W_in, b_in, Wl, bl, Wr, gamma, beta, W_out, b_out):
    src = edge_index[0]
    dst = edge_index[1]
    src2d = jnp.concatenate([src, _PAD_SRC]).reshape(_NW * _NB, _B)
    dst2d = jnp.concatenate([dst, _PAD_DST]).reshape(_NW * _NB, _B)
    z = jnp.zeros((_R, _D), jnp.float32)

    h0, t0 = _relu_proj(x, W_in.T, b_in.reshape(1, _D))
    degp = _cnt(dst2d, z, jnp.ones((_B, _D), jnp.float32))
    agg0 = _agg(t0, src2d, dst2d, z)
    h1, t1 = _step(h0, agg0, degp, Wl[0].T, bl[0].reshape(1, _D), Wr[0].T,
                   gamma[0].reshape(1, _D), beta[0].reshape(1, _D))
    agg1 = _agg(t1, src2d, dst2d, z)
    h2, t2 = _step(h1, agg1, degp, Wl[1].T, bl[1].reshape(1, _D), Wr[1].T,
                   gamma[1].reshape(1, _D), beta[1].reshape(1, _D))
    agg2 = _agg(t2, src2d, dst2d, z)
    wo = W_out.T
    part = _outproj3(h0, h1, h2, wo[:_D], wo[_D:2 * _D], wo[2 * _D:3 * _D],
                     b_out.reshape(1, _C))
    h3, = _step(h2, agg2, degp, Wl[2].T, bl[2].reshape(1, _D), Wr[2].T,
                gamma[2].reshape(1, _D), beta[2].reshape(1, _D), pack=False)
    out = _outproj_final(part, h3, wo[3 * _D:])
    return out


# speedup vs baseline: 1.1853x; 1.0342x over previous
import dataclasses
import functools

import numpy as np
import jax
import jax.numpy as jnp
from jax import lax
from jax.experimental import pallas as pl
from jax.experimental.pallas import tpu as pltpu
from jax.experimental.pallas import tpu_sc as plsc

_N = 10000
_E = 320000
_D = 128
_C = 64

_NC = 2
_NS = 16
_NW = _NC * _NS
_B = 128
_NB = 80
_EPAD = _NW * _NB * _B
_R = 10112
_RZ = _R // _NS

def _mesh():
    return plsc.VectorSubcoreMesh(core_axis_name="c", subcore_axis_name="s")


_CNB = 16
_NCH = _NB // _CNB


def _make_agg():
    _H = _B // 2
    out_type = jax.ShapeDtypeStruct((_NC, _R, _D), jnp.float32)
    scratch = [
        pltpu.VMEM((_CNB, _B), jnp.int32),
        pltpu.VMEM((2 * _CNB, _H), jnp.int32),
        pltpu.VMEM((_B, _D // 2), jnp.int32),
        pltpu.VMEM((_B, _D // 2), jnp.int32),
        pltpu.VMEM((_H, _D), jnp.float32),
        pltpu.VMEM((_H, _D), jnp.float32),
        pltpu.VMEM_SHARED((_R, _D), jnp.float32),
        pltpu.SemaphoreType.DMA,
        pltpu.SemaphoreType.DMA,
        pltpu.SemaphoreType.DMA,
        pltpu.SemaphoreType.DMA,
    ]

    def body(h_hbm, src_hbm, dsth_hbm, z_hbm, out_hbm,
             src_c, dst_c, b0, b1, f0, f1, acc_sh, gs0, gs1, ss0, ss1):
        c = lax.axis_index("c")
        s = lax.axis_index("s")
        w = s * _NC + c

        pltpu.sync_copy(z_hbm.at[pl.ds(s * _RZ, _RZ)],
                        acc_sh.at[pl.ds(s * _RZ, _RZ)])
        plsc.subcore_barrier()

        def gather(idx, buf, sem):
            pltpu.async_copy(h_hbm.at[idx], buf, sem)

        def gwait(buf, sem):
            pltpu.make_async_copy(h_hbm.at[src_c.at[0]], buf, sem).wait()

        stages = (f0, f1)
        ssems = (ss0, ss1)

        def scat(half, idx):
            pltpu.async_copy(stages[half], acc_sh.at[idx], ssems[half],
                             add=True)

        def swait(half):
            pltpu.make_async_copy(stages[half], acc_sh.at[dst_c.at[0]],
                                  ssems[half]).wait()

        def convert(bbuf, half):
            stage = stages[half]

            @plsc.parallel_loop(0, _H, unroll=2)
            def _(r):
                for k in range(_D // 32):
                    pair = plsc.bitcast(
                        bbuf[half * _H + r, pl.ds(16 * k, 16)], jnp.bfloat16)
                    lo, hi = plsc.unpack(
                        pair, format=plsc.PackFormat.INTERLEAVED,
                        preferred_element_type=jnp.float32)
                    stage[r, pl.ds(16 * k, 16)] = lo
                    stage[r, pl.ds(_D // 2 + 16 * k, 16)] = hi

        def do_batch(bbuf, gsem, first, idx0, idx1, nxt):
            gwait(bbuf, gsem)

            @pl.when(jnp.logical_not(first))
            def _():
                swait(0)

            convert(bbuf, 0)
            scat(0, idx0)

            @pl.when(jnp.logical_not(first))
            def _():
                swait(1)

            convert(bbuf, 1)
            nxt()
            scat(1, idx1)

        @pl.loop(0, _NCH)
        def _(ch):
            base = w * _NB + ch * _CNB
            pltpu.sync_copy(src_hbm.at[pl.ds(base, _CNB)], src_c)
            pltpu.sync_copy(dsth_hbm.at[pl.ds(2 * base, 2 * _CNB)], dst_c)
            gather(src_c.at[0], b0, gs0)
            gather(src_c.at[1], b1, gs1)

            @pl.loop(0, _CNB, step=2)
            def _(j):
                first = (ch == 0) & (j == 0)

                def nxt0():
                    @pl.when(j + 2 < _CNB)
                    def _():
                        gather(src_c.at[j + 2], b0, gs0)

                def nxt1():
                    @pl.when(j + 3 < _CNB)
                    def _():
                        gather(src_c.at[j + 3], b1, gs1)

                do_batch(b0, gs0, first, dst_c.at[2 * j],
                         dst_c.at[2 * j + 1], nxt0)
                do_batch(b1, gs1, first & False, dst_c.at[2 * j + 2],
                         dst_c.at[2 * j + 3], nxt1)

        swait(0)
        swait(1)
        plsc.subcore_barrier()
        pltpu.sync_copy(acc_sh.at[pl.ds(s * _RZ, _RZ)],
                        out_hbm.at[c, pl.ds(s * _RZ, _RZ)])

    return pl.kernel(body, out_type=out_type, mesh=_mesh(),
                     scratch_types=scratch, compiler_params=_sc_params())


def _sc_params():
    cp = pltpu.CompilerParams()
    fields = pltpu.CompilerParams.__dataclass_fields__
    if "needs_layout_passes" in fields:
        cp = dataclasses.replace(cp, needs_layout_passes=False)
    if "use_tc_tiling_on_sc" in fields:
        cp = dataclasses.replace(cp, use_tc_tiling_on_sc=False)
    return cp


def _make_cnt():
    out_type = jax.ShapeDtypeStruct((_NC, _R, _D), jnp.float32)
    scratch = [
        pltpu.VMEM((_CNB, _B), jnp.int32),
        pltpu.VMEM((_B, _D), jnp.float32),
        pltpu.VMEM_SHARED((_R, _D), jnp.float32),
        pltpu.SemaphoreType.DMA,
        pltpu.SemaphoreType.DMA,
    ]

    def body(dst_hbm, z_hbm, ones_hbm, out_hbm,
             dst_c, ones_v, acc_sh, ss0, ss1):
        c = lax.axis_index("c")
        s = lax.axis_index("s")
        w = s * _NC + c

        pltpu.sync_copy(z_hbm.at[pl.ds(s * _RZ, _RZ)],
                        acc_sh.at[pl.ds(s * _RZ, _RZ)])
        pltpu.sync_copy(ones_hbm, ones_v)
        plsc.subcore_barrier()

        def scat(idx, sem):
            pltpu.async_copy(ones_v, acc_sh.at[idx], sem, add=True)

        def swait(idx, sem):
            pltpu.make_async_copy(ones_v, acc_sh.at[idx], sem).wait()

        @pl.loop(0, _NCH)
        def _(ch):
            base = w * _NB + ch * _CNB
            pltpu.sync_copy(dst_hbm.at[pl.ds(base, _CNB)], dst_c)
            scat(dst_c.at[0], ss0)
            scat(dst_c.at[1], ss1)

            @pl.loop(2, _CNB, step=2)
            def _(j):
                swait(dst_c.at[j], ss0)
                scat(dst_c.at[j], ss0)
                swait(dst_c.at[j + 1], ss1)
                scat(dst_c.at[j + 1], ss1)

            swait(dst_c.at[_CNB - 2], ss0)
            swait(dst_c.at[_CNB - 1], ss1)

        plsc.subcore_barrier()
        pltpu.sync_copy(acc_sh.at[pl.ds(s * _RZ, _RZ)],
                        out_hbm.at[c, pl.ds(s * _RZ, _RZ)])

    return pl.kernel(body, out_type=out_type, mesh=_mesh(),
                     scratch_types=scratch, compiler_params=_sc_params())


_built = {}


def _agg(*args):
    if "agg" not in _built:
        _built["agg"] = _make_agg()
    return _built["agg"](*args)


def _cnt(*args):
    if "cnt" not in _built:
        _built["cnt"] = _make_cnt()
    return _built["cnt"](*args)

_BLK = 1000


def _pack_rows(hv):
    ua = lax.bitcast_convert_type(hv[:, :_D // 2], jnp.uint32)
    ub = lax.bitcast_convert_type(hv[:, _D // 2:], jnp.uint32)
    ra = ua + jnp.uint32(0x7FFF) + ((ua >> 16) & jnp.uint32(1))
    rb = ub + jnp.uint32(0x7FFF) + ((ub >> 16) & jnp.uint32(1))
    w = (ra >> 16) | (rb & jnp.uint32(0xFFFF0000))
    return lax.bitcast_convert_type(w, jnp.int32)


def _relu_proj(x, wt, b):
    def body(x_ref, w_ref, b_ref, o_ref, t_ref):
        h = jnp.maximum(
            jnp.dot(x_ref[...], w_ref[...],
                    preferred_element_type=jnp.float32) + b_ref[...], 0.0)
        o_ref[...] = h
        t_ref[...] = _pack_rows(h)

    return pl.pallas_call(
        body,
        grid=(_N // _BLK,),
        in_specs=[
            pl.BlockSpec((_BLK, _D), lambda i: (i, 0)),
            pl.BlockSpec((_D, _D), lambda i: (0, 0)),
            pl.BlockSpec((1, _D), lambda i: (0, 0)),
        ],
        out_specs=[pl.BlockSpec((_BLK, _D), lambda i: (i, 0)),
                   pl.BlockSpec((_BLK, _D // 2), lambda i: (i, 0))],
        out_shape=[jax.ShapeDtypeStruct((_N, _D), jnp.float32),
                   jax.ShapeDtypeStruct((_N, _D // 2), jnp.int32)],
    )(x, wt, b)


def _step(h, aggp, degp, wlt, blv, wrt, g, bt, pack=True):
    def body(h_ref, a_ref, d_ref, wl_ref, bl_ref, wr_ref, g_ref, bt_ref,
             o_ref, *t_ref):
        hv = h_ref[...]
        aggs = a_ref[0] + a_ref[1]
        degl = d_ref[0] + d_ref[1]
        deg = jnp.sum(degl, axis=1, keepdims=True) * (1.0 / _D)
        agg = aggs / jnp.maximum(deg, 1.0)
        h2 = (jnp.dot(agg, wl_ref[...], preferred_element_type=jnp.float32)
              + jnp.dot(hv, wr_ref[...], preferred_element_type=jnp.float32)
              + bl_ref[...])
        mu = jnp.mean(h2, axis=1, keepdims=True)
        dev = h2 - mu
        var = jnp.mean(dev * dev, axis=1, keepdims=True)
        hn = g_ref[...] * dev * lax.rsqrt(var + 1e-5) + bt_ref[...]
        hnew = jnp.maximum(hn, 0.0) + hv
        o_ref[...] = hnew
        if pack:
            t_ref[0][...] = _pack_rows(hnew)

    out_specs = [pl.BlockSpec((_BLK, _D), lambda i: (i, 0))]
    out_shape = [jax.ShapeDtypeStruct((_N, _D), jnp.float32)]
    if pack:
        out_specs.append(pl.BlockSpec((_BLK, _D // 2), lambda i: (i, 0)))
        out_shape.append(jax.ShapeDtypeStruct((_N, _D // 2), jnp.int32))
    return pl.pallas_call(
        body,
        grid=(_N // _BLK,),
        in_specs=[
            pl.BlockSpec((_BLK, _D), lambda i: (i, 0)),
            pl.BlockSpec((_NC, _BLK, _D), lambda i: (0, i, 0)),
            pl.BlockSpec((_NC, _BLK, _D), lambda i: (0, i, 0)),
            pl.BlockSpec((_D, _D), lambda i: (0, 0)),
            pl.BlockSpec((1, _D), lambda i: (0, 0)),
            pl.BlockSpec((_D, _D), lambda i: (0, 0)),
            pl.BlockSpec((1, _D), lambda i: (0, 0)),
            pl.BlockSpec((1, _D), lambda i: (0, 0)),
        ],
        out_specs=out_specs,
        out_shape=out_shape,
    )(h, aggp, degp, wlt, blv, wrt, g, bt)


def _outproj3(h0, h1, h2, w0, w1, w2, b):
    def body(h0_ref, h1_ref, h2_ref, w0_ref, w1_ref, w2_ref, b_ref, o_ref):
        o_ref[...] = (
            b_ref[...]
            + jnp.dot(h0_ref[...], w0_ref[...],
                      preferred_element_type=jnp.float32)
            + jnp.dot(h1_ref[...], w1_ref[...],
                      preferred_element_type=jnp.float32)
            + jnp.dot(h2_ref[...], w2_ref[...],
                      preferred_element_type=jnp.float32))

    hspec = pl.BlockSpec((_BLK, _D), lambda i: (i, 0))
    wspec = pl.BlockSpec((_D, _C), lambda i: (0, 0))
    return pl.pallas_call(
        body,
        grid=(_N // _BLK,),
        in_specs=[hspec, hspec, hspec, wspec, wspec, wspec,
                  pl.BlockSpec((1, _C), lambda i: (0, 0))],
        out_specs=pl.BlockSpec((_BLK, _C), lambda i: (i, 0)),
        out_shape=jax.ShapeDtypeStruct((_N, _C), jnp.float32),
    )(h0, h1, h2, w0, w1, w2, b)


def _outproj_final(part, h3, w3):
    def body(p_ref, h_ref, w_ref, o_ref):
        o_ref[...] = p_ref[...] + jnp.dot(
            h_ref[...], w_ref[...], preferred_element_type=jnp.float32)

    return pl.pallas_call(
        body,
        grid=(_N // _BLK,),
        in_specs=[
            pl.BlockSpec((_BLK, _C), lambda i: (i, 0)),
            pl.BlockSpec((_BLK, _D), lambda i: (i, 0)),
            pl.BlockSpec((_D, _C), lambda i: (0, 0)),
        ],
        out_specs=pl.BlockSpec((_BLK, _C), lambda i: (i, 0)),
        out_shape=jax.ShapeDtypeStruct((_N, _C), jnp.float32),
    )(part, h3, w3)


_PAD_I = np.arange(_EPAD - _E, dtype=np.int32)
_PAD_SRC = _PAD_I % _N
_PAD_DST = (_N + _PAD_I % (_R - _N)).astype(np.int32)


def kernel(x, edge_index, W_in, b_in, Wl, bl, Wr, gamma, beta, W_out, b_out):
    src = edge_index[0]
    dst = edge_index[1]
    src2d = jnp.concatenate([src, _PAD_SRC]).reshape(_NW * _NB, _B)
    dst2d = jnp.concatenate([dst, _PAD_DST]).reshape(_NW * _NB, _B)
    dst2dh = dst2d.reshape(_NW * _NB * 2, _B // 2)
    z = jnp.zeros((_R, _D), jnp.float32)

    h0, t0 = _relu_proj(x, W_in.T, b_in.reshape(1, _D))
    degp = _cnt(dst2d, z, jnp.ones((_B, _D), jnp.float32))
    agg0 = _agg(t0, src2d, dst2dh, z)
    h1, t1 = _step(h0, agg0, degp, Wl[0].T, bl[0].reshape(1, _D), Wr[0].T,
                   gamma[0].reshape(1, _D), beta[0].reshape(1, _D))
    agg1 = _agg(t1, src2d, dst2dh, z)
    h2, t2 = _step(h1, agg1, degp, Wl[1].T, bl[1].reshape(1, _D), Wr[1].T,
                   gamma[1].reshape(1, _D), beta[1].reshape(1, _D))
    agg2 = _agg(t2, src2d, dst2dh, z)
    wo = W_out.T
    part = _outproj3(h0, h1, h2, wo[:_D], wo[_D:2 * _D], wo[2 * _D:3 * _D],
                     b_out.reshape(1, _C))
    h3, = _step(h2, agg2, degp, Wl[2].T, bl[2].reshape(1, _D), Wr[2].T,
                gamma[2].reshape(1, _D), beta[2].reshape(1, _D), pack=False)
    out = _outproj_final(part, h3, wo[3 * _D:])
    return out
